# baseline (speedup 1.0000x reference)
import functools

import jax
import jax.numpy as jnp
from jax import lax
from jax.experimental import pallas as pl
from jax.experimental.pallas import tpu as pltpu
from jax.experimental.pallas import tpu_sc as plsc

B = 32
N = 3125
NP = 3200
PRE_K = 2000
POST_K = 300
ROWS_PAD = 304
ROW_W = 8
THRESH = 0.7
SZM1 = 254.0
NEG = -1.0e30
INT_MIN = -2147483648

SORT_N = 2048
SORT_V = SORT_N // 16
SORT_PAD = 2080
KEPT_PAD = 320
PHASE1_V = 32
PHASE1_N = PHASE1_V * 16


def _f32_sort_key(s):
    u = lax.bitcast_convert_type(s, jnp.int32)
    flipped = jnp.bitwise_xor(jnp.invert(u), INT_MIN)
    return jnp.where(u >= 0, u, flipped)


def _decode_body(pc_ref, pr_ref, anc_ref,
                 os_ref, ok_ref, ox1_ref, oy1_ref, ox2_ref, oy2_ref, othr_ref):
    s_parts = []
    x1_parts = []
    y1_parts = []
    x2_parts = []
    y2_parts = []
    for a in range(5):
        c0 = pc_ref[:, a, :]
        c1 = pc_ref[:, 5 + a, :]
        m = jnp.maximum(c0, c1)
        s = (c1 - m) - jnp.log(jnp.exp(c0 - m) + jnp.exp(c1 - m))
        s_parts.append(s)

        ax1 = anc_ref[0:1, a, :]
        ay1 = anc_ref[1:2, a, :]
        ax2 = anc_ref[2:3, a, :]
        ay2 = anc_ref[3:4, a, :]
        aw = ax2 - ax1 + 1.0
        ah = ay2 - ay1 + 1.0
        acx = ax1 + 0.5 * aw
        acy = ay1 + 0.5 * ah
        d0 = pr_ref[:, a, :]
        d1 = pr_ref[:, 5 + a, :]
        d2 = pr_ref[:, 10 + a, :]
        d3 = pr_ref[:, 15 + a, :]
        pcx = d0 * aw + acx
        pcy = d1 * ah + acy
        pw = jnp.exp(d2) * aw
        ph = jnp.exp(d3) * ah
        x1_parts.append(jnp.clip(pcx - 0.5 * pw, 0.0, SZM1))
        y1_parts.append(jnp.clip(pcy - 0.5 * ph, 0.0, SZM1))
        x2_parts.append(jnp.clip(pcx + 0.5 * pw, 0.0, SZM1))
        y2_parts.append(jnp.clip(pcy + 0.5 * ph, 0.0, SZM1))

    padf = jnp.full((B, NP - N), NEG, jnp.float32)
    s = jnp.concatenate(s_parts + [padf], axis=1)
    padz = jnp.zeros((B, NP - N), jnp.float32)
    x1 = jnp.concatenate(x1_parts + [padz], axis=1)
    y1 = jnp.concatenate(y1_parts + [padz], axis=1)
    x2 = jnp.concatenate(x2_parts + [padz], axis=1)
    y2 = jnp.concatenate(y2_parts + [padz], axis=1)

    key = _f32_sort_key(s)
    lo0 = jnp.min(key, axis=1, keepdims=True)
    hi0 = jnp.max(key, axis=1, keepdims=True) + 1

    def bs_step(_, lh):
        lo, hi = lh
        mid = lo + ((hi - lo) >> 1)
        cnt = jnp.sum((key >= mid).astype(jnp.int32), axis=1, keepdims=True)
        ge = cnt >= PRE_K
        return jnp.where(ge, mid, lo), jnp.where(ge, hi, mid)

    lo, _ = lax.fori_loop(0, 32, bs_step, (lo0, hi0))

    os_ref[:, :] = s
    ok_ref[:, :] = key
    ox1_ref[:, :] = x1
    oy1_ref[:, :] = y1
    ox2_ref[:, :] = x2
    oy2_ref[:, :] = y2
    othr_ref[:, :] = jnp.broadcast_to(lo, (B, 128))


def _sc_body(s_hbm, k_hbm, x1_hbm, y1_hbm, x2_hbm, y2_hbm, thr_hbm,
             out_hbm,
             s_sc, k_sc, x1_sc, y1_sc, x2_sc, y2_sc, thr_sc,
             skeys, sidx, kx1, ky1, kx2, ky2, ka, rows, sem):
    wid = lax.axis_index("s") * 2 + lax.axis_index("c")

    cps = [
        pltpu.async_copy(s_hbm.at[wid], s_sc, sem),
        pltpu.async_copy(k_hbm.at[wid], k_sc, sem),
        pltpu.async_copy(x1_hbm.at[wid], x1_sc, sem),
        pltpu.async_copy(y1_hbm.at[wid], y1_sc, sem),
        pltpu.async_copy(x2_hbm.at[wid], x2_sc, sem),
        pltpu.async_copy(y2_hbm.at[wid], y2_sc, sem),
        pltpu.async_copy(thr_hbm.at[wid], thr_sc, sem),
    ]

    iota16 = lax.iota(jnp.int32, 16)
    zero16i = jnp.zeros((16,), jnp.int32)
    lane0 = iota16 == 0
    lane8 = iota16 < 8

    def pre_sort(v, _):
        skeys[pl.ds(v * 32, 16)] = jnp.full((16,), INT_MIN, jnp.int32)
        skeys[pl.ds(v * 32 + 16, 16)] = jnp.full((16,), INT_MIN, jnp.int32)
        return 0

    lax.fori_loop(0, SORT_PAD // 32, pre_sort, 0)

    def pre_kept(v, _):
        off = v * 16
        kx1[pl.ds(off, 16)] = jnp.full((16,), 30000.0, jnp.float32)
        ky1[pl.ds(off, 16)] = jnp.full((16,), 30000.0, jnp.float32)
        kx2[pl.ds(off, 16)] = jnp.full((16,), 20000.0, jnp.float32)
        ky2[pl.ds(off, 16)] = jnp.full((16,), 20000.0, jnp.float32)
        ka[pl.ds(off, 16)] = jnp.full((16,), 1.0, jnp.float32)
        return 0

    def pre_rows(v, _):
        rows[pl.ds(v * 32, 16)] = jnp.zeros((16,), jnp.float32)
        rows[pl.ds(v * 32 + 16, 16)] = jnp.zeros((16,), jnp.float32)
        return 0

    def reset_nms_state():
        lax.fori_loop(0, KEPT_PAD // 16, pre_kept, 0)
        lax.fori_loop(0, (ROWS_PAD * ROW_W) // 32, pre_rows, 0)

    reset_nms_state()

    for cp in cps:
        cp.wait()

    thr = thr_sc[pl.ds(0, 16)]

    def comp1(g, cnt):
        k = k_sc[pl.ds(g * 16, 16)]
        msk = k >= thr
        idx = iota16 + jnp.full((16,), g * 16, jnp.int32)
        mi = msk.astype(jnp.int32)
        pos = jnp.full((16,), cnt, jnp.int32) + plsc.cumsum(mi) - 1
        pos = jnp.minimum(pos, SORT_PAD - 1)
        plsc.store_scatter(skeys, [pos], k, mask=msk)
        plsc.store_scatter(sidx, [pos], idx, mask=msk)
        return cnt + jnp.sum(mi)

    def comp2(g, cnt):
        cnt = comp1(2 * g, cnt)
        return comp1(2 * g + 1, cnt)

    cnt = lax.fori_loop(0, NP // 32, comp2, jnp.int32(0))
    cnt = jnp.minimum(cnt, SORT_N)

    def vsort_one(v):
        off = v * 16
        k = skeys[pl.ds(off, 16)]
        p = sidx[pl.ds(off, 16)]
        ks, ps = plsc.sort_key_val(k, p, descending=True)
        skeys[pl.ds(off, 16)] = ks
        sidx[pl.ds(off, 16)] = ps

    def emit_vsort_pass(nv):
        def b2(v, _):
            vsort_one(2 * v)
            vsort_one(2 * v + 1)
            return 0

        lax.fori_loop(0, nv // 2, b2, 0)

    def emit_ce(i, d):
        oa = i * 16
        ob = (i + d) * 16
        ka_ = skeys[pl.ds(oa, 16)]
        kb_ = skeys[pl.ds(ob, 16)]
        pa_ = sidx[pl.ds(oa, 16)]
        pb_ = sidx[pl.ds(ob, 16)]
        m_ = ka_ >= kb_
        skeys[pl.ds(oa, 16)] = jnp.where(m_, ka_, kb_)
        skeys[pl.ds(ob, 16)] = jnp.where(m_, kb_, ka_)
        sidx[pl.ds(oa, 16)] = jnp.where(m_, pa_, pb_)
        sidx[pl.ds(ob, 16)] = jnp.where(m_, pb_, pa_)

    def emit_sort(nv):
        emit_vsort_pass(nv)
        r = 1
        while r < nv:
            nm = nv // (2 * r)

            def revb(mi, _, r=r):
                base = mi * 2 * r + r
                if r == 1:
                    off = base * 16
                    skeys[pl.ds(off, 16)] = jnp.flip(skeys[pl.ds(off, 16)])
                    sidx[pl.ds(off, 16)] = jnp.flip(sidx[pl.ds(off, 16)])
                else:
                    for j in range(r // 2):
                        o1 = (base + j) * 16
                        o2 = (base + r - 1 - j) * 16
                        ka_ = skeys[pl.ds(o1, 16)]
                        kb_ = skeys[pl.ds(o2, 16)]
                        pa_ = sidx[pl.ds(o1, 16)]
                        pb_ = sidx[pl.ds(o2, 16)]
                        skeys[pl.ds(o1, 16)] = jnp.flip(kb_)
                        skeys[pl.ds(o2, 16)] = jnp.flip(ka_)
                        sidx[pl.ds(o1, 16)] = jnp.flip(pb_)
                        sidx[pl.ds(o2, 16)] = jnp.flip(pa_)
                return 0

            lax.fori_loop(0, nm, revb, 0)

            d = r
            while d >= 1:
                def ce2(p, _, d=d):
                    p0 = 2 * p
                    p1 = 2 * p + 1
                    emit_ce((p0 // d) * (2 * d) + (p0 % d), d)
                    emit_ce((p1 // d) * (2 * d) + (p1 % d), d)
                    return 0

                lax.fori_loop(0, nv // 4, ce2, 0)
                d //= 2

            emit_vsort_pass(nv)
            r *= 2

    def emit_tie_fix(nv):
        for parity in (0, 1, 0, 1):
            def tie_fix(t, _, parity=parity):
                ev = jnp.full((16,), t * 32 + parity, jnp.int32) + 2 * iota16
                od = ev + 1
                ka_ = plsc.load_gather(skeys, [ev])
                kb_ = plsc.load_gather(skeys, [od])
                pa_ = plsc.load_gather(sidx, [ev])
                pb_ = plsc.load_gather(sidx, [od])
                swap = (ka_ == kb_) & (pa_ > pb_)
                plsc.store_scatter(sidx, [ev], jnp.where(swap, pb_, pa_))
                plsc.store_scatter(sidx, [od], jnp.where(swap, pa_, pb_))
                return 0

            lax.fori_loop(0, nv // 2, tie_fix, 0)

    jcap = jnp.minimum(cnt, PRE_K)

    def run_nms(jlimit):
        def nms_cond(st):
            j, nk = st
            return jnp.logical_and(j < jlimit, nk < POST_K)

        def nms_body(st):
            j, nk = st
            jv = jnp.full((16,), j, jnp.int32)
            gi = plsc.load_gather(sidx, [jv])
            cx1 = plsc.load_gather(x1_sc, [gi])
            cy1 = plsc.load_gather(y1_sc, [gi])
            cx2 = plsc.load_gather(x2_sc, [gi])
            cy2 = plsc.load_gather(y2_sc, [gi])
            cs = plsc.load_gather(s_sc, [gi])
            carea = (cx2 - cx1 + 1.0) * (cy2 - cy1 + 1.0)

            def iou_chunk(c):
                off = c * 16
                kx1v = kx1[pl.ds(off, 16)]
                ky1v = ky1[pl.ds(off, 16)]
                kx2v = kx2[pl.ds(off, 16)]
                ky2v = ky2[pl.ds(off, 16)]
                kav = ka[pl.ds(off, 16)]
                xx1 = jnp.maximum(kx1v, cx1)
                yy1 = jnp.maximum(ky1v, cy1)
                xx2 = jnp.minimum(kx2v, cx2)
                yy2 = jnp.minimum(ky2v, cy2)
                iw = jnp.maximum(xx2 - xx1 + 1.0, 0.0)
                ih = jnp.maximum(yy2 - yy1 + 1.0, 0.0)
                inter = iw * ih
                iou = inter / ((carea + kav) - inter)
                return (iou > THRESH).astype(jnp.int32)

            def inner2(c, supp):
                supp = supp | iou_chunk(2 * c)
                return supp | iou_chunk(2 * c + 1)

            nch2 = (nk + 31) >> 5
            supp = lax.fori_loop(0, nch2, inner2, zero16i)
            keep_i = 1 - jnp.minimum(jnp.max(supp), 1)
            keepv = jnp.full((16,), keep_i, jnp.int32) > 0

            nkv = jnp.full((16,), nk, jnp.int32)
            m_app = lane0 & keepv
            plsc.store_scatter(kx1, [nkv], cx1, mask=m_app)
            plsc.store_scatter(ky1, [nkv], cy1, mask=m_app)
            plsc.store_scatter(kx2, [nkv], cx2, mask=m_app)
            plsc.store_scatter(ky2, [nkv], cy2, mask=m_app)
            plsc.store_scatter(ka, [nkv], carea, mask=m_app)

            rowv = jnp.where(iota16 == 0, cs,
                   jnp.where(iota16 == 1, cx1,
                   jnp.where(iota16 == 2, cy1,
                   jnp.where(iota16 == 3, cx2,
                   jnp.where(iota16 == 4, cy2,
                             jnp.zeros((16,), jnp.float32))))))
            ridx = nkv * ROW_W + iota16
            plsc.store_scatter(rows, [ridx], rowv, mask=lane8 & keepv)
            return j + 1, nk + keep_i

        return lax.while_loop(nms_cond, nms_body, (jnp.int32(0), jnp.int32(0)))

    emit_sort(PHASE1_V)
    emit_tie_fix(PHASE1_V)
    _, nk1 = run_nms(jnp.minimum(jcap, PHASE1_N))

    @pl.when(jnp.logical_and(nk1 < POST_K, jcap > PHASE1_N))
    def _phase2():
        reset_nms_state()
        emit_sort(SORT_V)
        emit_tie_fix(SORT_V)
        run_nms(jcap)

    pltpu.sync_copy(rows, out_hbm.at[wid])


@jax.jit
def kernel(pred_cls, pred_reg, anchors):
    anc = anchors.T.reshape(4, 5, 625)

    f32 = jnp.float32
    i32 = jnp.int32
    dec_sh = [
        jax.ShapeDtypeStruct((B, NP), f32),
        jax.ShapeDtypeStruct((B, NP), i32),
        jax.ShapeDtypeStruct((B, NP), f32),
        jax.ShapeDtypeStruct((B, NP), f32),
        jax.ShapeDtypeStruct((B, NP), f32),
        jax.ShapeDtypeStruct((B, NP), f32),
        jax.ShapeDtypeStruct((B, 128), i32),
    ]
    s, key, x1, y1, x2, y2, thr = pl.pallas_call(
        _decode_body, out_shape=dec_sh)(
            pred_cls.reshape(B, 10, 625), pred_reg.reshape(B, 20, 625), anc)

    mesh = plsc.VectorSubcoreMesh(core_axis_name="c", subcore_axis_name="s")
    sc = functools.partial(
        pl.kernel,
        mesh=mesh,
        compiler_params=pltpu.CompilerParams(needs_layout_passes=False),
        out_type=jax.ShapeDtypeStruct((B, ROWS_PAD * ROW_W), f32),
        scratch_types=[
            pltpu.VMEM((NP,), f32),
            pltpu.VMEM((NP,), i32),
            pltpu.VMEM((NP,), f32),
            pltpu.VMEM((NP,), f32),
            pltpu.VMEM((NP,), f32),
            pltpu.VMEM((NP,), f32),
            pltpu.VMEM((128,), i32),
            pltpu.VMEM((SORT_PAD,), i32),
            pltpu.VMEM((SORT_PAD,), i32),
            pltpu.VMEM((KEPT_PAD,), f32),
            pltpu.VMEM((KEPT_PAD,), f32),
            pltpu.VMEM((KEPT_PAD,), f32),
            pltpu.VMEM((KEPT_PAD,), f32),
            pltpu.VMEM((KEPT_PAD,), f32),
            pltpu.VMEM((ROWS_PAD * ROW_W,), f32),
            pltpu.SemaphoreType.DMA,
        ],
    )(_sc_body)
    rows = sc(s, key, x1, y1, x2, y2, thr)
    return rows.reshape(B, ROWS_PAD, ROW_W)[:, :POST_K, :5]


# speedup vs baseline: 1.5098x; 1.5098x over previous
import functools

import jax
import jax.numpy as jnp
from jax import lax
from jax.experimental import pallas as pl
from jax.experimental.pallas import tpu as pltpu
from jax.experimental.pallas import tpu_sc as plsc

B = 32
N = 3125
NP = 3200
PRE_K = 2000
POST_K = 300
ROWS_PAD = 304
ROW_W = 8
THRESH = 0.7
SZM1 = 254.0
NEG = -1.0e30
INT_MIN = -2147483648

KEPT_PAD = 320
PH1_K = 512
A_CAP_V = 64
B_BASE_V = 64
B_CAP = 2048
FULL_V = 256
SORT_PAD = FULL_V * 16 + 32


def _f32_sort_key(s):
    u = lax.bitcast_convert_type(s, jnp.int32)
    flipped = jnp.bitwise_xor(jnp.invert(u), INT_MIN)
    return jnp.where(u >= 0, u, flipped)


def _decode_body(pc_ref, pr_ref, anc_ref,
                 os_ref, ok_ref, ox1_ref, oy1_ref, ox2_ref, oy2_ref, othr_ref):
    s_parts = []
    x1_parts = []
    y1_parts = []
    x2_parts = []
    y2_parts = []
    for a in range(5):
        c0 = pc_ref[:, a, :]
        c1 = pc_ref[:, 5 + a, :]
        m = jnp.maximum(c0, c1)
        s = (c1 - m) - jnp.log(jnp.exp(c0 - m) + jnp.exp(c1 - m))
        s_parts.append(s)

        ax1 = anc_ref[0:1, a, :]
        ay1 = anc_ref[1:2, a, :]
        ax2 = anc_ref[2:3, a, :]
        ay2 = anc_ref[3:4, a, :]
        aw = ax2 - ax1 + 1.0
        ah = ay2 - ay1 + 1.0
        acx = ax1 + 0.5 * aw
        acy = ay1 + 0.5 * ah
        d0 = pr_ref[:, a, :]
        d1 = pr_ref[:, 5 + a, :]
        d2 = pr_ref[:, 10 + a, :]
        d3 = pr_ref[:, 15 + a, :]
        pcx = d0 * aw + acx
        pcy = d1 * ah + acy
        pw = jnp.exp(d2) * aw
        ph = jnp.exp(d3) * ah
        x1_parts.append(jnp.clip(pcx - 0.5 * pw, 0.0, SZM1))
        y1_parts.append(jnp.clip(pcy - 0.5 * ph, 0.0, SZM1))
        x2_parts.append(jnp.clip(pcx + 0.5 * pw, 0.0, SZM1))
        y2_parts.append(jnp.clip(pcy + 0.5 * ph, 0.0, SZM1))

    padf = jnp.full((B, NP - N), NEG, jnp.float32)
    s = jnp.concatenate(s_parts + [padf], axis=1)
    padz = jnp.zeros((B, NP - N), jnp.float32)
    x1 = jnp.concatenate(x1_parts + [padz], axis=1)
    y1 = jnp.concatenate(y1_parts + [padz], axis=1)
    x2 = jnp.concatenate(x2_parts + [padz], axis=1)
    y2 = jnp.concatenate(y2_parts + [padz], axis=1)

    key = _f32_sort_key(s)
    lo0 = jnp.min(key, axis=1, keepdims=True)
    hi0 = jnp.max(key, axis=1, keepdims=True) + 1

    def kth_key(k):
        def bs_step(_, lh):
            lo, hi = lh
            mid = lo + ((hi - lo) >> 1)
            cnt = jnp.sum((key >= mid).astype(jnp.int32), axis=1, keepdims=True)
            ge = cnt >= k
            return jnp.where(ge, mid, lo), jnp.where(ge, hi, mid)

        lo, _ = lax.fori_loop(0, 32, bs_step, (lo0, hi0))
        return lo

    lo = kth_key(PRE_K)
    lo512 = kth_key(PH1_K)

    os_ref[:, :] = s
    ok_ref[:, :] = key
    ox1_ref[:, :] = x1
    oy1_ref[:, :] = y1
    ox2_ref[:, :] = x2
    oy2_ref[:, :] = y2
    othr_ref[:, :] = jnp.concatenate(
        [jnp.broadcast_to(lo, (B, 64)), jnp.broadcast_to(lo512, (B, 64))],
        axis=1)


def _sc_body(s_hbm, k_hbm, x1_hbm, y1_hbm, x2_hbm, y2_hbm, thr_hbm,
             out_hbm,
             s_sc, k_sc, x1_sc, y1_sc, x2_sc, y2_sc, thr_sc,
             skeys, sidx, kx1, ky1, kx2, ky2, ka, rows, sem):
    wid = lax.axis_index("s") * 2 + lax.axis_index("c")

    cps = [
        pltpu.async_copy(s_hbm.at[wid], s_sc, sem),
        pltpu.async_copy(k_hbm.at[wid], k_sc, sem),
        pltpu.async_copy(x1_hbm.at[wid], x1_sc, sem),
        pltpu.async_copy(y1_hbm.at[wid], y1_sc, sem),
        pltpu.async_copy(x2_hbm.at[wid], x2_sc, sem),
        pltpu.async_copy(y2_hbm.at[wid], y2_sc, sem),
        pltpu.async_copy(thr_hbm.at[wid], thr_sc, sem),
    ]

    iota16 = lax.iota(jnp.int32, 16)
    zero16i = jnp.zeros((16,), jnp.int32)
    lane0 = iota16 == 0
    lane8 = iota16 < 8

    def pre_sort(v, _):
        skeys[pl.ds(v * 32, 16)] = jnp.full((16,), INT_MIN, jnp.int32)
        skeys[pl.ds(v * 32 + 16, 16)] = jnp.full((16,), INT_MIN, jnp.int32)
        return 0

    lax.fori_loop(0, SORT_PAD // 32, pre_sort, 0)

    def pre_kept(v, _):
        off = v * 16
        kx1[pl.ds(off, 16)] = jnp.full((16,), 30000.0, jnp.float32)
        ky1[pl.ds(off, 16)] = jnp.full((16,), 30000.0, jnp.float32)
        kx2[pl.ds(off, 16)] = jnp.full((16,), 20000.0, jnp.float32)
        ky2[pl.ds(off, 16)] = jnp.full((16,), 20000.0, jnp.float32)
        ka[pl.ds(off, 16)] = jnp.full((16,), 1.0, jnp.float32)
        return 0

    def pre_rows(v, _):
        rows[pl.ds(v * 32, 16)] = jnp.zeros((16,), jnp.float32)
        rows[pl.ds(v * 32 + 16, 16)] = jnp.zeros((16,), jnp.float32)
        return 0

    def reset_nms_state():
        lax.fori_loop(0, KEPT_PAD // 16, pre_kept, 0)
        lax.fori_loop(0, (ROWS_PAD * ROW_W) // 32, pre_rows, 0)

    reset_nms_state()

    for cp in cps:
        cp.wait()

    thr = thr_sc[pl.ds(0, 16)]
    thr512 = thr_sc[pl.ds(64, 16)]

    def comp1(g, st):
        ca, cb = st
        k = k_sc[pl.ds(g * 16, 16)]
        idx = iota16 + jnp.full((16,), g * 16, jnp.int32)
        ma = k >= thr512
        mb = (k >= thr) & jnp.logical_not(ma)
        mia = ma.astype(jnp.int32)
        mib = mb.astype(jnp.int32)
        pa = jnp.full((16,), ca, jnp.int32) + plsc.cumsum(mia) - 1
        pa = jnp.minimum(pa, A_CAP_V * 16 - 1)
        pb = jnp.full((16,), B_BASE_V * 16 + cb, jnp.int32) + plsc.cumsum(mib) - 1
        pb = jnp.minimum(pb, B_BASE_V * 16 + B_CAP - 1)
        plsc.store_scatter(skeys, [pa], k, mask=ma)
        plsc.store_scatter(sidx, [pa], idx, mask=ma)
        plsc.store_scatter(skeys, [pb], k, mask=mb)
        plsc.store_scatter(sidx, [pb], idx, mask=mb)
        return ca + jnp.sum(mia), cb + jnp.sum(mib)

    def comp2(g, st):
        return comp1(2 * g + 1, comp1(2 * g, st))

    cnt_a, cnt_b = lax.fori_loop(
        0, NP // 32, comp2, (jnp.int32(0), jnp.int32(0)))
    cnt_a = jnp.minimum(cnt_a, A_CAP_V * 16)
    cnt_b = jnp.minimum(cnt_b, B_CAP)

    def vsort_one(v):
        off = v * 16
        k = skeys[pl.ds(off, 16)]
        p = sidx[pl.ds(off, 16)]
        ks, ps = plsc.sort_key_val(k, p, descending=True)
        skeys[pl.ds(off, 16)] = ks
        sidx[pl.ds(off, 16)] = ps

    def emit_vsort_pass(nv):
        def b2(v, _):
            vsort_one(2 * v)
            vsort_one(2 * v + 1)
            return 0

        lax.fori_loop(0, nv // 2, b2, 0)

    def emit_ce(i, d):
        oa = i * 16
        ob = (i + d) * 16
        ka_ = skeys[pl.ds(oa, 16)]
        kb_ = skeys[pl.ds(ob, 16)]
        pa_ = sidx[pl.ds(oa, 16)]
        pb_ = sidx[pl.ds(ob, 16)]
        m_ = ka_ >= kb_
        skeys[pl.ds(oa, 16)] = jnp.where(m_, ka_, kb_)
        skeys[pl.ds(ob, 16)] = jnp.where(m_, kb_, ka_)
        sidx[pl.ds(oa, 16)] = jnp.where(m_, pa_, pb_)
        sidx[pl.ds(ob, 16)] = jnp.where(m_, pb_, pa_)

    def emit_sort(nv):
        emit_vsort_pass(nv)
        r = 1
        while r < nv:
            nm = nv // (2 * r)

            def revb(mi, _, r=r):
                base = mi * 2 * r + r
                if r == 1:
                    off = base * 16
                    skeys[pl.ds(off, 16)] = jnp.flip(skeys[pl.ds(off, 16)])
                    sidx[pl.ds(off, 16)] = jnp.flip(sidx[pl.ds(off, 16)])
                else:
                    for j in range(r // 2):
                        o1 = (base + j) * 16
                        o2 = (base + r - 1 - j) * 16
                        ka_ = skeys[pl.ds(o1, 16)]
                        kb_ = skeys[pl.ds(o2, 16)]
                        pa_ = sidx[pl.ds(o1, 16)]
                        pb_ = sidx[pl.ds(o2, 16)]
                        skeys[pl.ds(o1, 16)] = jnp.flip(kb_)
                        skeys[pl.ds(o2, 16)] = jnp.flip(ka_)
                        sidx[pl.ds(o1, 16)] = jnp.flip(pb_)
                        sidx[pl.ds(o2, 16)] = jnp.flip(pa_)
                return 0

            lax.fori_loop(0, nm, revb, 0)

            d = r
            while d >= 1:
                def ce2(p, _, d=d):
                    p0 = 2 * p
                    p1 = 2 * p + 1
                    emit_ce((p0 // d) * (2 * d) + (p0 % d), d)
                    emit_ce((p1 // d) * (2 * d) + (p1 % d), d)
                    return 0

                lax.fori_loop(0, nv // 4, ce2, 0)
                d //= 2

            emit_vsort_pass(nv)
            r *= 2

    def emit_tie_fix(nv):
        for parity in (0, 1, 0, 1):
            def tie_fix(t, _, parity=parity):
                ev = jnp.full((16,), t * 32 + parity, jnp.int32) + 2 * iota16
                od = ev + 1
                ka_ = plsc.load_gather(skeys, [ev])
                kb_ = plsc.load_gather(skeys, [od])
                pa_ = plsc.load_gather(sidx, [ev])
                pb_ = plsc.load_gather(sidx, [od])
                swap = (ka_ == kb_) & (pa_ > pb_)
                plsc.store_scatter(sidx, [ev], jnp.where(swap, pb_, pa_))
                plsc.store_scatter(sidx, [od], jnp.where(swap, pa_, pb_))
                return 0

            lax.fori_loop(0, nv // 2, tie_fix, 0)

    jcap = jnp.minimum(cnt_a + cnt_b, PRE_K)

    def run_nms(jlimit):
        def nms_cond(st):
            j, nk = st
            return jnp.logical_and(j < jlimit, nk < POST_K)

        def nms_body(st):
            j, nk = st
            jv = jnp.full((16,), j, jnp.int32)
            gi = plsc.load_gather(sidx, [jv])
            cx1 = plsc.load_gather(x1_sc, [gi])
            cy1 = plsc.load_gather(y1_sc, [gi])
            cx2 = plsc.load_gather(x2_sc, [gi])
            cy2 = plsc.load_gather(y2_sc, [gi])
            cs = plsc.load_gather(s_sc, [gi])
            carea = (cx2 - cx1 + 1.0) * (cy2 - cy1 + 1.0)

            def iou_chunk(c):
                off = c * 16
                kx1v = kx1[pl.ds(off, 16)]
                ky1v = ky1[pl.ds(off, 16)]
                kx2v = kx2[pl.ds(off, 16)]
                ky2v = ky2[pl.ds(off, 16)]
                kav = ka[pl.ds(off, 16)]
                xx1 = jnp.maximum(kx1v, cx1)
                yy1 = jnp.maximum(ky1v, cy1)
                xx2 = jnp.minimum(kx2v, cx2)
                yy2 = jnp.minimum(ky2v, cy2)
                iw = jnp.maximum(xx2 - xx1 + 1.0, 0.0)
                ih = jnp.maximum(yy2 - yy1 + 1.0, 0.0)
                inter = iw * ih
                iou = inter / ((carea + kav) - inter)
                return (iou > THRESH).astype(jnp.int32)

            def inner2(c, supp):
                supp = supp | iou_chunk(2 * c)
                return supp | iou_chunk(2 * c + 1)

            nch2 = (nk + 31) >> 5
            supp = lax.fori_loop(0, nch2, inner2, zero16i)
            keep_i = 1 - jnp.minimum(jnp.max(supp), 1)
            keepv = jnp.full((16,), keep_i, jnp.int32) > 0

            nkv = jnp.full((16,), nk, jnp.int32)
            m_app = lane0 & keepv
            plsc.store_scatter(kx1, [nkv], cx1, mask=m_app)
            plsc.store_scatter(ky1, [nkv], cy1, mask=m_app)
            plsc.store_scatter(kx2, [nkv], cx2, mask=m_app)
            plsc.store_scatter(ky2, [nkv], cy2, mask=m_app)
            plsc.store_scatter(ka, [nkv], carea, mask=m_app)

            rowv = jnp.where(iota16 == 0, cs,
                   jnp.where(iota16 == 1, cx1,
                   jnp.where(iota16 == 2, cy1,
                   jnp.where(iota16 == 3, cx2,
                   jnp.where(iota16 == 4, cy2,
                             jnp.zeros((16,), jnp.float32))))))
            ridx = nkv * ROW_W + iota16
            plsc.store_scatter(rows, [ridx], rowv, mask=lane8 & keepv)
            return j + 1, nk + keep_i

        return lax.while_loop(nms_cond, nms_body, (jnp.int32(0), jnp.int32(0)))

    emit_sort(A_CAP_V)
    emit_tie_fix(A_CAP_V)
    _, nk1 = run_nms(jnp.minimum(jcap, cnt_a))

    @pl.when(jnp.logical_and(nk1 < POST_K, jcap > cnt_a))
    def _phase2():
        reset_nms_state()
        emit_sort(FULL_V)
        emit_tie_fix(FULL_V)
        run_nms(jcap)

    pltpu.sync_copy(rows, out_hbm.at[wid])


@jax.jit
def kernel(pred_cls, pred_reg, anchors):
    anc = anchors.T.reshape(4, 5, 625)

    f32 = jnp.float32
    i32 = jnp.int32
    dec_sh = [
        jax.ShapeDtypeStruct((B, NP), f32),
        jax.ShapeDtypeStruct((B, NP), i32),
        jax.ShapeDtypeStruct((B, NP), f32),
        jax.ShapeDtypeStruct((B, NP), f32),
        jax.ShapeDtypeStruct((B, NP), f32),
        jax.ShapeDtypeStruct((B, NP), f32),
        jax.ShapeDtypeStruct((B, 128), i32),
    ]
    s, key, x1, y1, x2, y2, thr = pl.pallas_call(
        _decode_body, out_shape=dec_sh)(
            pred_cls.reshape(B, 10, 625), pred_reg.reshape(B, 20, 625), anc)

    mesh = plsc.VectorSubcoreMesh(core_axis_name="c", subcore_axis_name="s")
    sc = functools.partial(
        pl.kernel,
        mesh=mesh,
        compiler_params=pltpu.CompilerParams(needs_layout_passes=False),
        out_type=jax.ShapeDtypeStruct((B, ROWS_PAD * ROW_W), f32),
        scratch_types=[
            pltpu.VMEM((NP,), f32),
            pltpu.VMEM((NP,), i32),
            pltpu.VMEM((NP,), f32),
            pltpu.VMEM((NP,), f32),
            pltpu.VMEM((NP,), f32),
            pltpu.VMEM((NP,), f32),
            pltpu.VMEM((128,), i32),
            pltpu.VMEM((SORT_PAD,), i32),
            pltpu.VMEM((SORT_PAD,), i32),
            pltpu.VMEM((KEPT_PAD,), f32),
            pltpu.VMEM((KEPT_PAD,), f32),
            pltpu.VMEM((KEPT_PAD,), f32),
            pltpu.VMEM((KEPT_PAD,), f32),
            pltpu.VMEM((KEPT_PAD,), f32),
            pltpu.VMEM((ROWS_PAD * ROW_W,), f32),
            pltpu.SemaphoreType.DMA,
        ],
    )(_sc_body)
    rows = sc(s, key, x1, y1, x2, y2, thr)
    return rows.reshape(B, ROWS_PAD, ROW_W)[:, :POST_K, :5]
